# Initial kernel scaffold; baseline (speedup 1.0000x reference)
import jax
import jax.numpy as jnp
from jax.experimental import pallas as pl


def kernel(x, params):
    raise NotImplementedError("write your pallas kernel here")


# speedup vs baseline: 1.0071x; 1.0071x over previous
import jax
import jax.numpy as jnp
from jax.experimental import pallas as pl
from jax.experimental.pallas import tpu as pltpu

_LAYER_CFG = [(512, 32, 0.4, [64, 64, 128]), (128, 64, 0.8, [128, 128, 256])]
_BN_EPS = 1e-5


def _fps_body(start_ref, coord_ref, out_ref):
    b = pl.program_id(0)
    R, C = coord_ref.shape[1], coord_ref.shape[2]
    n = R * C
    iota = (jax.lax.broadcasted_iota(jnp.int32, (R, C), 0) * C
            + jax.lax.broadcasted_iota(jnp.int32, (R, C), 1))
    c0 = coord_ref[0]
    c1 = coord_ref[1]
    c2 = coord_ref[2]
    S = out_ref.shape[1]

    def body(i, carry):
        dist, far = carry
        out_ref[b, i] = far
        sel = (iota == far).astype(jnp.float32)
        f0 = jnp.sum(c0 * sel)
        f1 = jnp.sum(c1 * sel)
        f2 = jnp.sum(c2 * sel)
        d = jnp.sqrt((c0 - f0) ** 2 + (c1 - f1) ** 2 + (c2 - f2) ** 2 + 1e-12)
        dist = jnp.minimum(dist, d)
        m = jnp.max(dist)
        far = jnp.min(jnp.where(dist == m, iota, n)).astype(jnp.int32)
        return dist, far

    jax.lax.fori_loop(
        0, S, body,
        (jnp.full((R, C), jnp.inf, dtype=jnp.float32), start_ref[b]))


def _fps_indices(coord, start, S):
    B, _, N = coord.shape
    C = 128
    R = N // C
    coord_r = coord.reshape(B, 3, R, C)
    return pl.pallas_call(
        _fps_body,
        grid=(B,),
        in_specs=[
            pl.BlockSpec(memory_space=pltpu.SMEM),
            pl.BlockSpec((None, 3, R, C), lambda b: (b, 0, 0, 0)),
        ],
        out_specs=pl.BlockSpec(memory_space=pltpu.SMEM),
        out_shape=jax.ShapeDtypeStruct((B, S), jnp.int32),
    )(start, coord_r)


def _fps(batch, S, key):
    B, F, N = batch.shape
    coord = jax.lax.stop_gradient(batch[:, :3, :])
    farthest = jax.random.randint(key, (B,), 0, N, dtype=jnp.int32)
    sampled = _fps_indices(coord, farthest, S)
    idx = jnp.broadcast_to(sampled[:, None, :], (B, F, S))
    return jnp.take_along_axis(batch, idx, axis=2)


def _ball_query(batch, radius, Q, centers):
    B, F, N = batch.shape
    S = centers.shape[2]
    p = jax.lax.stop_gradient(batch[:, :3, :])
    c = jax.lax.stop_gradient(centers[:, :3, :])
    c2 = jnp.sum(c * c, axis=1)
    p2 = jnp.sum(p * p, axis=1)
    d2 = jnp.maximum(c2[:, :, None] + p2[:, None, :] - 2.0 * jnp.einsum('bcs,bcn->bsn', c, p), 0.0)
    dist = jnp.sqrt(d2 + 1e-12)
    _, idx = jax.lax.top_k(-dist, Q)
    mask = dist < radius
    has_valid = jnp.any(mask, axis=2)
    first_valid = jnp.argmax(mask.astype(jnp.int32), axis=2)
    first_valid = jnp.where(has_valid, first_valid, 0).astype(idx.dtype)
    fv = jnp.broadcast_to(first_valid[:, :, None], (B, S, Q))
    masked = jnp.take_along_axis(mask, idx, axis=2)
    idx = jnp.where(masked, idx, fv)
    flat = jnp.broadcast_to(idx.reshape(B, 1, S * Q), (B, F, S * Q))
    qr = jnp.take_along_axis(batch, flat, axis=2).reshape(B, F, S, Q)
    qr = qr.at[:, :3, :, :].add(-centers[:, :3, :, None])
    return qr


def _mlp(x, params, li, n_layers):
    for i in range(n_layers):
        w = params['conv_w_%d_%d' % (li, i)]
        b = params['conv_b_%d_%d' % (li, i)]
        x = jnp.einsum('oc,bcsq->bosq', w, x) + b[None, :, None, None]
        mu = jnp.mean(x, axis=(0, 2, 3), keepdims=True)
        var = jnp.mean((x - mu) ** 2, axis=(0, 2, 3), keepdims=True)
        x = (x - mu) / jnp.sqrt(var + _BN_EPS)
        x = x * params['bn_g_%d_%d' % (li, i)][None, :, None, None] + params['bn_b_%d_%d' % (li, i)][None, :, None, None]
        x = jnp.maximum(x, 0.0)
    return x


def kernel(x, params):
    key = jax.random.key(42)
    for li, (S, Q, R, mlp_layers) in enumerate(_LAYER_CFG):
        key, kf = jax.random.split(key)
        centers = _fps(x, S, kf)
        grouped = _ball_query(x, R, Q, centers)
        feat = _mlp(grouped, params, li, len(mlp_layers))
        x = jnp.max(feat, axis=-1)
    x = jnp.mean(x, axis=-1)
    h = jnp.maximum(x @ params['g_w0'].T + params['g_b0'], 0.0)
    h = jnp.maximum(h @ params['g_w1'].T + params['g_b1'], 0.0)
    mu = h @ params['mu_w'].T + params['mu_b']
    lv = h @ params['lv_w'].T + params['lv_b']
    key, ke = jax.random.split(key)
    eps = jax.random.normal(ke, mu.shape, dtype=mu.dtype)
    z = mu + eps * jnp.exp(0.5 * lv)
    return z, mu, lv
